# baseline (speedup 1.0000x reference)
import functools

import jax
import jax.numpy as jnp
from jax import lax
from jax.experimental import pallas as pl
from jax.experimental.pallas import tpu as pltpu
from jax.experimental.pallas import tpu_sc as plsc

N = 10000
E = 320000
D = 128
G = 8
H = 64

NW = 32
CHUNK = 128
GRP = 4
NCH = 80
EPAD = NW * NCH * CHUNK
NPAD = 10240
RPT = NPAD // 16
RB = 1000
NB = N // RB

_f32 = jnp.float32


def _sc_mesh():
    return plsc.VectorSubcoreMesh(core_axis_name="c", subcore_axis_name="s")


_SC_PARAMS = pltpu.CompilerParams(use_tc_tiling_on_sc=False)


def _sc_deg(srcs):
    @functools.partial(
        pl.kernel,
        out_type=jax.ShapeDtypeStruct((2, NPAD), _f32),
        mesh=_sc_mesh(),
        compiler_params=_SC_PARAMS,
        scratch_types=[
            pltpu.VMEM((NCH, CHUNK), jnp.int32),
            pltpu.VMEM((CHUNK,), _f32),
            pltpu.VMEM((RPT,), _f32),
            pltpu.VMEM_SHARED((NPAD,), _f32),
        ],
    )
    def k(srcs_hbm, out_hbm, si_v, ones_v, zb_v, dacc_sh):
        cid = lax.axis_index("c")
        sid = lax.axis_index("s")
        wid = sid * 2 + cid
        pltpu.sync_copy(srcs_hbm.at[wid], si_v)
        for t in range(CHUNK // 16):
            ones_v[pl.ds(t * 16, 16)] = jnp.ones((16,), _f32)

        def zinit(i, carry):
            zb_v[pl.ds(i * 16, 16)] = jnp.zeros((16,), _f32)
            return carry

        lax.fori_loop(0, RPT // 16, zinit, 0)
        pltpu.sync_copy(zb_v, dacc_sh.at[pl.ds(sid * RPT, RPT)])
        plsc.subcore_barrier()

        def body(j, carry):
            pltpu.sync_copy(ones_v, dacc_sh.at[si_v.at[j]], add=True)
            return carry

        lax.fori_loop(0, NCH, body, 0)
        plsc.subcore_barrier()
        pltpu.sync_copy(dacc_sh.at[pl.ds(sid * RPT, RPT)],
                        out_hbm.at[cid, pl.ds(sid * RPT, RPT)])

    return k(srcs)


def _sc_matvec(w, srcg, dsts):
    @functools.partial(
        pl.kernel,
        out_type=jax.ShapeDtypeStruct((2, NPAD, H), _f32),
        mesh=_sc_mesh(),
        compiler_params=_SC_PARAMS,
        scratch_types=[
            pltpu.VMEM((NCH, CHUNK), jnp.int32),
            pltpu.VMEM((NCH, CHUNK), jnp.int32),
            pltpu.VMEM((2 * GRP, CHUNK, H), _f32),
            pltpu.VMEM_SHARED((NPAD, H), _f32),
            pltpu.SemaphoreType.DMA,
            pltpu.SemaphoreType.DMA,
            pltpu.SemaphoreType.DMA,
            pltpu.SemaphoreType.DMA,
        ],
    )
    def k(w_hbm, srcg_hbm, dsts_hbm, out_hbm, sg_v, ds_v, rows, acc_sh,
          gsem0, gsem1, ssem0, ssem1):
        cid = lax.axis_index("c")
        sid = lax.axis_index("s")
        wid = sid * 2 + cid
        gsems = (gsem0, gsem1)
        ssems = (ssem0, ssem1)
        pltpu.async_copy(srcg_hbm.at[wid], sg_v, gsem0)
        pltpu.async_copy(dsts_hbm.at[wid], ds_v, gsem1)

        def zinit(i, carry):
            r = i // (H // 16)
            c = i % (H // 16)
            rows[0, r, pl.ds(c * 16, 16)] = jnp.zeros((16,), _f32)
            return carry

        lax.fori_loop(0, CHUNK * (H // 16), zinit, 0)
        for t in range(RPT // CHUNK):
            pltpu.sync_copy(rows.at[0],
                            acc_sh.at[pl.ds(sid * RPT + t * CHUNK, CHUNK)])
        pltpu.make_async_copy(srcg_hbm.at[wid], sg_v, gsem0).wait()
        pltpu.make_async_copy(dsts_hbm.at[wid], ds_v, gsem1).wait()
        plsc.subcore_barrier()

        def fire_gathers(j0, bank, sem):
            for b in range(GRP):
                pltpu.async_copy(w_hbm.at[sg_v.at[j0 + b]],
                                 rows.at[bank * GRP + b], sem)

        def wait_gathers(j0, bank, sem):
            for b in range(GRP):
                pltpu.make_async_copy(w_hbm.at[sg_v.at[j0 + b]],
                                      rows.at[bank * GRP + b], sem).wait()

        def fire_scatters(j0, bank, sem):
            for b in range(GRP):
                pltpu.async_copy(rows.at[bank * GRP + b],
                                 acc_sh.at[ds_v.at[j0 + b]], sem, add=True)

        def wait_scatters(j0, bank, sem):
            for b in range(GRP):
                pltpu.make_async_copy(rows.at[bank * GRP + b],
                                      acc_sh.at[ds_v.at[j0 + b]], sem).wait()

        ngrp = NCH // GRP
        fire_gathers(0, 0, gsem0)

        def body(g, carry):
            t0 = 2 * g
            wait_gathers(t0 * GRP, 0, gsem0)
            fire_scatters(t0 * GRP, 0, ssem0)

            @pl.when(g > 0)
            def _():
                wait_scatters((t0 - 1) * GRP, 1, ssem1)

            fire_gathers((t0 + 1) * GRP, 1, gsem1)
            wait_gathers((t0 + 1) * GRP, 1, gsem1)
            fire_scatters((t0 + 1) * GRP, 1, ssem1)
            wait_scatters(t0 * GRP, 0, ssem0)

            @pl.when(t0 + 2 < ngrp)
            def _():
                fire_gathers((t0 + 2) * GRP, 0, gsem0)

            return carry

        lax.fori_loop(0, ngrp // 2, body, 0)
        wait_scatters((ngrp - 1) * GRP, 1, ssem1)
        plsc.subcore_barrier()
        pltpu.sync_copy(acc_sh.at[pl.ds(sid * RPT, RPT)],
                        out_hbm.at[cid, pl.ds(sid * RPT, RPT)])

    return k(w, srcg, dsts)


def _tc_pre(x, wcat, dg0, dg1):
    def body(x_ref, w_ref, dg0_ref, dg1_ref, xw_ref, dis_ref, wa_ref):
        xw = jnp.dot(x_ref[...], w_ref[...], preferred_element_type=_f32)
        deg = dg0_ref[...] + dg1_ref[...]
        dis = jnp.where(deg > 0, lax.rsqrt(jnp.maximum(deg, 1e-12)), 0.0)
        xw_ref[...] = xw
        dis_ref[...] = dis
        wa_ref[...] = dis * xw[:, 2 * H:3 * H]

    return pl.pallas_call(
        body,
        grid=(NB,),
        in_specs=[
            pl.BlockSpec((RB, D), lambda i: (i, 0)),
            pl.BlockSpec((D, 3 * H), lambda i: (0, 0)),
            pl.BlockSpec((RB, 1), lambda i: (i, 0)),
            pl.BlockSpec((RB, 1), lambda i: (i, 0)),
        ],
        out_specs=[
            pl.BlockSpec((RB, 3 * H), lambda i: (i, 0)),
            pl.BlockSpec((RB, 1), lambda i: (i, 0)),
            pl.BlockSpec((RB, H), lambda i: (i, 0)),
        ],
        out_shape=[
            jax.ShapeDtypeStruct((N, 3 * H), _f32),
            jax.ShapeDtypeStruct((N, 1), _f32),
            jax.ShapeDtypeStruct((N, H), _f32),
        ],
    )(x, wcat, dg0, dg1)


def _tc_mid(xw, dis, p):
    def body(xw_ref, dis_ref, p_ref, wc_ref):
        ps = p_ref[0] + p_ref[1]
        dis = dis_ref[...]
        wc_ref[...] = dis * (xw_ref[:, H:2 * H] - 2.0 * dis * ps)

    return pl.pallas_call(
        body,
        grid=(NB,),
        in_specs=[
            pl.BlockSpec((RB, 3 * H), lambda i: (i, 0)),
            pl.BlockSpec((RB, 1), lambda i: (i, 0)),
            pl.BlockSpec((2, RB, H), lambda i: (0, i, 0)),
        ],
        out_specs=pl.BlockSpec((RB, H), lambda i: (i, 0)),
        out_shape=jax.ShapeDtypeStruct((N, H), _f32),
    )(xw, dis, p)


def _tc_layer(xw, dis, p, b, w2cat):
    def body(xw_ref, dis_ref, p_ref, b_ref, w2_ref, xw2_ref, wa2_ref):
        ps = p_ref[0] + p_ref[1]
        dis = dis_ref[...]
        h = jnp.maximum(
            xw_ref[:, 0:H] - xw_ref[:, 2 * H:3 * H] - dis * ps + b_ref[...], 0.0)
        xw2 = jnp.dot(h, w2_ref[...], preferred_element_type=_f32)
        xw2_ref[...] = xw2
        wa2_ref[...] = dis * xw2[:, 2 * H:3 * H]

    return pl.pallas_call(
        body,
        grid=(NB,),
        in_specs=[
            pl.BlockSpec((RB, 3 * H), lambda i: (i, 0)),
            pl.BlockSpec((RB, 1), lambda i: (i, 0)),
            pl.BlockSpec((2, RB, H), lambda i: (0, i, 0)),
            pl.BlockSpec((1, H), lambda i: (0, 0)),
            pl.BlockSpec((H, 3 * H), lambda i: (0, 0)),
        ],
        out_specs=[
            pl.BlockSpec((RB, 3 * H), lambda i: (i, 0)),
            pl.BlockSpec((RB, H), lambda i: (i, 0)),
        ],
        out_shape=[
            jax.ShapeDtypeStruct((N, 3 * H), _f32),
            jax.ShapeDtypeStruct((N, H), _f32),
        ],
    )(xw, dis, p, b, w2cat)


def _tc_final(xw2, dis, p, b, batch2d, wfc, bfc):
    def body(xw_ref, dis_ref, p_ref, b_ref, bat_ref, wfc_ref, bfc_ref, out_ref,
             s_sum, s_cnt):
        i = pl.program_id(0)

        @pl.when(i == 0)
        def _():
            s_sum[...] = jnp.zeros((G, H), _f32)
            s_cnt[...] = jnp.zeros((G, 128), _f32)

        ps = p_ref[0] + p_ref[1]
        dis = dis_ref[...]
        h2 = jnp.maximum(
            xw_ref[:, 0:H] - xw_ref[:, 2 * H:3 * H] - dis * ps + b_ref[...], 0.0)
        bat = bat_ref[...][:, 0]
        onehot = (lax.broadcasted_iota(jnp.int32, (G, RB), 0)
                  == bat[None, :]).astype(_f32)
        s_sum[...] += jnp.dot(onehot, h2, preferred_element_type=_f32)
        s_cnt[...] += jnp.broadcast_to(
            jnp.sum(onehot, axis=1)[:, None], (G, 128))

        @pl.when(i == NB - 1)
        def _():
            pooled = s_sum[...] / jnp.maximum(s_cnt[:, 0:1], 1.0)
            out_ref[...] = (jnp.dot(pooled, wfc_ref[...],
                                    preferred_element_type=_f32) + bfc_ref[...])

    return pl.pallas_call(
        body,
        grid=(NB,),
        in_specs=[
            pl.BlockSpec((RB, 3 * H), lambda i: (i, 0)),
            pl.BlockSpec((RB, 1), lambda i: (i, 0)),
            pl.BlockSpec((2, RB, H), lambda i: (0, i, 0)),
            pl.BlockSpec((1, H), lambda i: (0, 0)),
            pl.BlockSpec((RB, 1), lambda i: (i, 0)),
            pl.BlockSpec((H, 1), lambda i: (0, 0)),
            pl.BlockSpec((1, 1), lambda i: (0, 0)),
        ],
        out_specs=pl.BlockSpec((G, 1), lambda i: (0, 0)),
        out_shape=jax.ShapeDtypeStruct((G, 1), _f32),
        scratch_shapes=[
            pltpu.VMEM((G, H), _f32),
            pltpu.VMEM((G, 128), _f32),
        ],
    )(xw2, dis, p, b, batch2d, wfc, bfc)


def kernel(x, edge_index, batch, W1, b1, W2, b2, Wfc, bfc):
    src = edge_index[0].astype(jnp.int32)
    dst = edge_index[1].astype(jnp.int32)

    npad = EPAD - E
    padi = jnp.arange(npad, dtype=jnp.int32)
    srcg = jnp.concatenate([src, padi % N]).reshape(NW, NCH, CHUNK)
    srcs = jnp.concatenate([src, N + (padi % 16)]).reshape(NW, NCH, CHUNK)
    dsts = jnp.concatenate([dst, N + (padi % 16)]).reshape(NW, NCH, CHUNK)

    w1cat = jnp.concatenate([W1[0], W1[1], W1[2]], axis=1)
    w2cat = jnp.concatenate([W2[0], W2[1], W2[2]], axis=1)
    b1r = b1.reshape(1, H)
    b2r = b2.reshape(1, H)
    bfcr = bfc.reshape(1, 1)
    batch2d = batch.astype(jnp.int32).reshape(N, 1)

    degp = _sc_deg(srcs)
    dg0 = degp[0].reshape(NPAD, 1)
    dg1 = degp[1].reshape(NPAD, 1)

    xw1, dis, wa1 = _tc_pre(x, w1cat, dg0, dg1)
    p1 = _sc_matvec(wa1, srcg, dsts)
    wc1 = _tc_mid(xw1, dis, p1)
    p2 = _sc_matvec(wc1, srcg, dsts)
    xw2, wa2 = _tc_layer(xw1, dis, p2, b1r, w2cat)
    p3 = _sc_matvec(wa2, srcg, dsts)
    wc2 = _tc_mid(xw2, dis, p3)
    p4 = _sc_matvec(wc2, srcg, dsts)
    out = _tc_final(xw2, dis, p4, b2r, batch2d, Wfc, bfcr)
    return out[:, 0]


# speedup vs baseline: 1.0015x; 1.0015x over previous
import functools

import jax
import jax.numpy as jnp
from jax import lax
from jax.experimental import pallas as pl
from jax.experimental.pallas import tpu as pltpu
from jax.experimental.pallas import tpu_sc as plsc

N = 10000
E = 320000
D = 128
G = 8
H = 64

NW = 32
CHUNK = 128
GRP = 4
NCH = 80
EPAD = NW * NCH * CHUNK
NPAD = 10240
RPT = NPAD // 16
RB = 1000
NB = N // RB

_f32 = jnp.float32


def _sc_mesh():
    return plsc.VectorSubcoreMesh(core_axis_name="c", subcore_axis_name="s")


_SC_PARAMS = pltpu.CompilerParams(use_tc_tiling_on_sc=False)


def _sc_deg(srcs):
    @functools.partial(
        pl.kernel,
        out_type=jax.ShapeDtypeStruct((2, NPAD), _f32),
        mesh=_sc_mesh(),
        compiler_params=_SC_PARAMS,
        scratch_types=[
            pltpu.VMEM((NCH, CHUNK), jnp.int32),
            pltpu.VMEM((CHUNK,), _f32),
            pltpu.VMEM((RPT,), _f32),
            pltpu.VMEM_SHARED((NPAD,), _f32),
        ],
    )
    def k(srcs_hbm, out_hbm, si_v, ones_v, zb_v, dacc_sh):
        cid = lax.axis_index("c")
        sid = lax.axis_index("s")
        wid = sid * 2 + cid
        pltpu.sync_copy(srcs_hbm.at[wid], si_v)
        for t in range(CHUNK // 16):
            ones_v[pl.ds(t * 16, 16)] = jnp.ones((16,), _f32)

        def zinit(i, carry):
            zb_v[pl.ds(i * 16, 16)] = jnp.zeros((16,), _f32)
            return carry

        lax.fori_loop(0, RPT // 16, zinit, 0)
        pltpu.sync_copy(zb_v, dacc_sh.at[pl.ds(sid * RPT, RPT)])
        plsc.subcore_barrier()

        def body(j, carry):
            pltpu.sync_copy(ones_v, dacc_sh.at[si_v.at[j]], add=True)
            return carry

        lax.fori_loop(0, NCH, body, 0)
        plsc.subcore_barrier()
        pltpu.sync_copy(dacc_sh.at[pl.ds(sid * RPT, RPT)],
                        out_hbm.at[cid, pl.ds(sid * RPT, RPT)])

    return k(srcs)


def _sc_matvec(w, srcg, dsts):
    @functools.partial(
        pl.kernel,
        out_type=jax.ShapeDtypeStruct((2, NPAD, H), _f32),
        mesh=_sc_mesh(),
        compiler_params=_SC_PARAMS,
        scratch_types=[
            pltpu.VMEM((NCH, CHUNK), jnp.int32),
            pltpu.VMEM((NCH, CHUNK), jnp.int32),
            pltpu.VMEM((2 * GRP, CHUNK, H), _f32),
            pltpu.VMEM_SHARED((NPAD, H), _f32),
            pltpu.SemaphoreType.DMA,
            pltpu.SemaphoreType.DMA,
            pltpu.SemaphoreType.DMA,
            pltpu.SemaphoreType.DMA,
        ],
    )
    def k(w_hbm, srcg_hbm, dsts_hbm, out_hbm, sg_v, ds_v, rows, acc_sh,
          gsem0, gsem1, ssem0, ssem1):
        cid = lax.axis_index("c")
        sid = lax.axis_index("s")
        wid = sid * 2 + cid
        gsems = (gsem0, gsem1)
        ssems = (ssem0, ssem1)
        pltpu.async_copy(srcg_hbm.at[wid], sg_v, gsem0)
        pltpu.async_copy(dsts_hbm.at[wid], ds_v, gsem1)

        def zinit(i, carry):
            r = i // (H // 16)
            c = i % (H // 16)
            rows[0, r, pl.ds(c * 16, 16)] = jnp.zeros((16,), _f32)
            return carry

        lax.fori_loop(0, CHUNK * (H // 16), zinit, 0)
        for t in range(RPT // CHUNK):
            pltpu.sync_copy(rows.at[0],
                            acc_sh.at[pl.ds(sid * RPT + t * CHUNK, CHUNK)])
        pltpu.make_async_copy(srcg_hbm.at[wid], sg_v, gsem0).wait()
        pltpu.make_async_copy(dsts_hbm.at[wid], ds_v, gsem1).wait()
        plsc.subcore_barrier()

        def fire_gathers(j0, bank, sem):
            for b in range(GRP):
                pltpu.async_copy(w_hbm.at[sg_v.at[j0 + b]],
                                 rows.at[bank * GRP + b], sem)

        def wait_gathers(j0, bank, sem):
            for b in range(GRP):
                pltpu.make_async_copy(w_hbm.at[sg_v.at[j0 + b]],
                                      rows.at[bank * GRP + b], sem).wait()

        def fire_scatters(j0, bank, sem):
            for b in range(GRP):
                pltpu.async_copy(rows.at[bank * GRP + b],
                                 acc_sh.at[ds_v.at[j0 + b]], sem, add=True)

        def wait_scatters(j0, bank, sem):
            for b in range(GRP):
                pltpu.make_async_copy(rows.at[bank * GRP + b],
                                      acc_sh.at[ds_v.at[j0 + b]], sem).wait()

        ngrp = NCH // GRP
        fire_gathers(0, 0, gsem0)

        def body(g, carry):
            t0 = 2 * g
            wait_gathers(t0 * GRP, 0, gsem0)
            fire_scatters(t0 * GRP, 0, ssem0)

            @pl.when(g > 0)
            def _():
                wait_scatters((t0 - 1) * GRP, 1, ssem1)

            fire_gathers((t0 + 1) * GRP, 1, gsem1)
            wait_gathers((t0 + 1) * GRP, 1, gsem1)
            fire_scatters((t0 + 1) * GRP, 1, ssem1)
            wait_scatters(t0 * GRP, 0, ssem0)

            @pl.when(t0 + 2 < ngrp)
            def _():
                fire_gathers((t0 + 2) * GRP, 0, gsem0)

            return carry

        lax.fori_loop(0, ngrp // 2, body, 0)
        wait_scatters((ngrp - 1) * GRP, 1, ssem1)
        plsc.subcore_barrier()
        pltpu.sync_copy(acc_sh.at[pl.ds(sid * RPT, RPT)],
                        out_hbm.at[cid, pl.ds(sid * RPT, RPT)])

    return k(w, srcg, dsts)


def _tc_pre(x, wcat, degpt):
    def body(x_ref, w_ref, dg_ref, xw_ref, wa_ref):
        xw = jnp.dot(x_ref[...], w_ref[...], preferred_element_type=_f32)
        deg = dg_ref[:, 0:1] + dg_ref[:, 1:2]
        dis = jnp.where(deg > 0, lax.rsqrt(jnp.maximum(deg, 1e-12)), 0.0)
        xw_ref[...] = xw
        wa_ref[...] = dis * xw[:, 2 * H:3 * H]

    return pl.pallas_call(
        body,
        grid=(NB,),
        in_specs=[
            pl.BlockSpec((RB, D), lambda i: (i, 0)),
            pl.BlockSpec((D, 3 * H), lambda i: (0, 0)),
            pl.BlockSpec((RB, 2), lambda i: (i, 0)),
        ],
        out_specs=[
            pl.BlockSpec((RB, 3 * H), lambda i: (i, 0)),
            pl.BlockSpec((RB, H), lambda i: (i, 0)),
        ],
        out_shape=[
            jax.ShapeDtypeStruct((N, 3 * H), _f32),
            jax.ShapeDtypeStruct((N, H), _f32),
        ],
    )(x, wcat, degpt)


def _tc_mid(xw, degpt, p):
    def body(xw_ref, dg_ref, p_ref, wc_ref):
        ps = p_ref[0] + p_ref[1]
        deg = dg_ref[:, 0:1] + dg_ref[:, 1:2]
        dis = jnp.where(deg > 0, lax.rsqrt(jnp.maximum(deg, 1e-12)), 0.0)
        wc_ref[...] = dis * (xw_ref[:, H:2 * H] - 2.0 * dis * ps)

    return pl.pallas_call(
        body,
        grid=(NB,),
        in_specs=[
            pl.BlockSpec((RB, 3 * H), lambda i: (i, 0)),
            pl.BlockSpec((RB, 2), lambda i: (i, 0)),
            pl.BlockSpec((2, RB, H), lambda i: (0, i, 0)),
        ],
        out_specs=pl.BlockSpec((RB, H), lambda i: (i, 0)),
        out_shape=jax.ShapeDtypeStruct((N, H), _f32),
    )(xw, degpt, p)


def _tc_layer(xw, degpt, p, b, w2cat):
    def body(xw_ref, dg_ref, p_ref, b_ref, w2_ref, xw2_ref, wa2_ref):
        ps = p_ref[0] + p_ref[1]
        deg = dg_ref[:, 0:1] + dg_ref[:, 1:2]
        dis = jnp.where(deg > 0, lax.rsqrt(jnp.maximum(deg, 1e-12)), 0.0)
        h = jnp.maximum(
            xw_ref[:, 0:H] - xw_ref[:, 2 * H:3 * H] - dis * ps + b_ref[...], 0.0)
        xw2 = jnp.dot(h, w2_ref[...], preferred_element_type=_f32)
        xw2_ref[...] = xw2
        wa2_ref[...] = dis * xw2[:, 2 * H:3 * H]

    return pl.pallas_call(
        body,
        grid=(NB,),
        in_specs=[
            pl.BlockSpec((RB, 3 * H), lambda i: (i, 0)),
            pl.BlockSpec((RB, 2), lambda i: (i, 0)),
            pl.BlockSpec((2, RB, H), lambda i: (0, i, 0)),
            pl.BlockSpec((1, H), lambda i: (0, 0)),
            pl.BlockSpec((H, 3 * H), lambda i: (0, 0)),
        ],
        out_specs=[
            pl.BlockSpec((RB, 3 * H), lambda i: (i, 0)),
            pl.BlockSpec((RB, H), lambda i: (i, 0)),
        ],
        out_shape=[
            jax.ShapeDtypeStruct((N, 3 * H), _f32),
            jax.ShapeDtypeStruct((N, H), _f32),
        ],
    )(xw, degpt, p, b, w2cat)


def _tc_final(xw2, aux, p, b, wfc, bfc):
    def body(xw_ref, aux_ref, p_ref, b_ref, wfc_ref, bfc_ref, out_ref,
             s_sum, s_cnt):
        i = pl.program_id(0)

        @pl.when(i == 0)
        def _():
            s_sum[...] = jnp.zeros((G, H), _f32)
            s_cnt[...] = jnp.zeros((G, 128), _f32)

        ps = p_ref[0] + p_ref[1]
        deg = aux_ref[:, 0:1] + aux_ref[:, 1:2]
        dis = jnp.where(deg > 0, lax.rsqrt(jnp.maximum(deg, 1e-12)), 0.0)
        h2 = jnp.maximum(
            xw_ref[:, 0:H] - xw_ref[:, 2 * H:3 * H] - dis * ps + b_ref[...], 0.0)
        bat = aux_ref[:, 2].astype(jnp.int32)
        onehot = (lax.broadcasted_iota(jnp.int32, (G, RB), 0)
                  == bat[None, :]).astype(_f32)
        s_sum[...] += jnp.dot(onehot, h2, preferred_element_type=_f32)
        s_cnt[...] += jnp.broadcast_to(
            jnp.sum(onehot, axis=1)[:, None], (G, 128))

        @pl.when(i == NB - 1)
        def _():
            pooled = s_sum[...] / jnp.maximum(s_cnt[:, 0:1], 1.0)
            out_ref[...] = (jnp.dot(pooled, wfc_ref[...],
                                    preferred_element_type=_f32) + bfc_ref[...])

    return pl.pallas_call(
        body,
        grid=(NB,),
        in_specs=[
            pl.BlockSpec((RB, 3 * H), lambda i: (i, 0)),
            pl.BlockSpec((RB, 3), lambda i: (i, 0)),
            pl.BlockSpec((2, RB, H), lambda i: (0, i, 0)),
            pl.BlockSpec((1, H), lambda i: (0, 0)),
            pl.BlockSpec((H, 1), lambda i: (0, 0)),
            pl.BlockSpec((1, 1), lambda i: (0, 0)),
        ],
        out_specs=pl.BlockSpec((G, 1), lambda i: (0, 0)),
        out_shape=jax.ShapeDtypeStruct((G, 1), _f32),
        scratch_shapes=[
            pltpu.VMEM((G, H), _f32),
            pltpu.VMEM((G, 128), _f32),
        ],
    )(xw2, aux, p, b, wfc, bfc)


def kernel(x, edge_index, batch, W1, b1, W2, b2, Wfc, bfc):
    src = edge_index[0].astype(jnp.int32)
    dst = edge_index[1].astype(jnp.int32)

    npad = EPAD - E
    padi = jnp.arange(npad, dtype=jnp.int32)
    srcg = jnp.concatenate([src, padi % N]).reshape(NW, NCH, CHUNK)
    srcs = jnp.concatenate([src, N + (padi % 16)]).reshape(NW, NCH, CHUNK)
    dsts = jnp.concatenate([dst, N + (padi % 16)]).reshape(NW, NCH, CHUNK)

    w1cat = jnp.concatenate([W1[0], W1[1], W1[2]], axis=1)
    w2cat = jnp.concatenate([W2[0], W2[1], W2[2]], axis=1)
    b1r = b1.reshape(1, H)
    b2r = b2.reshape(1, H)
    bfcr = bfc.reshape(1, 1)

    degp = _sc_deg(srcs)
    degpt = degp.T[:N]
    aux = jnp.concatenate(
        [degpt, batch.astype(_f32).reshape(N, 1)], axis=1)

    xw1, wa1 = _tc_pre(x, w1cat, degpt)
    p1 = _sc_matvec(wa1, srcg, dsts)
    wc1 = _tc_mid(xw1, degpt, p1)
    p2 = _sc_matvec(wc1, srcg, dsts)
    xw2, wa2 = _tc_layer(xw1, degpt, p2, b1r, w2cat)
    p3 = _sc_matvec(wa2, srcg, dsts)
    wc2 = _tc_mid(xw2, degpt, p3)
    p4 = _sc_matvec(wc2, srcg, dsts)
    out = _tc_final(xw2, aux, p4, b2r, Wfc, bfcr)
    return out[:, 0]


# speedup vs baseline: 1.0206x; 1.0190x over previous
import functools

import jax
import jax.numpy as jnp
from jax import lax
from jax.experimental import pallas as pl
from jax.experimental.pallas import tpu as pltpu
from jax.experimental.pallas import tpu_sc as plsc

N = 10000
E = 320000
D = 128
G = 8
H = 64

NW = 32
CHUNK = 128
GRP = 4
NCH = 80
EPAD = NW * NCH * CHUNK
NPAD = 10240
RPT = NPAD // 16
RB = 2000
NB = N // RB

_f32 = jnp.float32


def _sc_mesh():
    return plsc.VectorSubcoreMesh(core_axis_name="c", subcore_axis_name="s")


_SC_PARAMS = pltpu.CompilerParams(use_tc_tiling_on_sc=False)


def _sc_deg(srcs):
    @functools.partial(
        pl.kernel,
        out_type=jax.ShapeDtypeStruct((2, NPAD), _f32),
        mesh=_sc_mesh(),
        compiler_params=_SC_PARAMS,
        scratch_types=[
            pltpu.VMEM((NCH, CHUNK), jnp.int32),
            pltpu.VMEM((CHUNK,), _f32),
            pltpu.VMEM((RPT,), _f32),
            pltpu.VMEM_SHARED((NPAD,), _f32),
        ],
    )
    def k(srcs_hbm, out_hbm, si_v, ones_v, zb_v, dacc_sh):
        cid = lax.axis_index("c")
        sid = lax.axis_index("s")
        wid = sid * 2 + cid
        pltpu.sync_copy(srcs_hbm.at[wid], si_v)
        for t in range(CHUNK // 16):
            ones_v[pl.ds(t * 16, 16)] = jnp.ones((16,), _f32)

        def zinit(i, carry):
            zb_v[pl.ds(i * 16, 16)] = jnp.zeros((16,), _f32)
            return carry

        lax.fori_loop(0, RPT // 16, zinit, 0)
        pltpu.sync_copy(zb_v, dacc_sh.at[pl.ds(sid * RPT, RPT)])
        plsc.subcore_barrier()

        def body(j, carry):
            pltpu.sync_copy(ones_v, dacc_sh.at[si_v.at[j]], add=True)
            return carry

        lax.fori_loop(0, NCH, body, 0)
        plsc.subcore_barrier()
        pltpu.sync_copy(dacc_sh.at[pl.ds(sid * RPT, RPT)],
                        out_hbm.at[cid, pl.ds(sid * RPT, RPT)])

    return k(srcs)


def _sc_matvec(w, srcg, dsts):
    @functools.partial(
        pl.kernel,
        out_type=jax.ShapeDtypeStruct((2, NPAD, H), _f32),
        mesh=_sc_mesh(),
        compiler_params=_SC_PARAMS,
        scratch_types=[
            pltpu.VMEM((NCH, CHUNK), jnp.int32),
            pltpu.VMEM((NCH, CHUNK), jnp.int32),
            pltpu.VMEM((2 * GRP, CHUNK, H), _f32),
            pltpu.VMEM_SHARED((NPAD, H), _f32),
            pltpu.SemaphoreType.DMA,
            pltpu.SemaphoreType.DMA,
            pltpu.SemaphoreType.DMA,
            pltpu.SemaphoreType.DMA,
        ],
    )
    def k(w_hbm, srcg_hbm, dsts_hbm, out_hbm, sg_v, ds_v, rows, acc_sh,
          gsem0, gsem1, ssem0, ssem1):
        cid = lax.axis_index("c")
        sid = lax.axis_index("s")
        wid = sid * 2 + cid
        gsems = (gsem0, gsem1)
        ssems = (ssem0, ssem1)
        pltpu.async_copy(srcg_hbm.at[wid], sg_v, gsem0)
        pltpu.async_copy(dsts_hbm.at[wid], ds_v, gsem1)

        def zinit(i, carry):
            r = i // (H // 16)
            c = i % (H // 16)
            rows[0, r, pl.ds(c * 16, 16)] = jnp.zeros((16,), _f32)
            return carry

        lax.fori_loop(0, CHUNK * (H // 16), zinit, 0)
        for t in range(RPT // CHUNK):
            pltpu.sync_copy(rows.at[0],
                            acc_sh.at[pl.ds(sid * RPT + t * CHUNK, CHUNK)])
        pltpu.make_async_copy(srcg_hbm.at[wid], sg_v, gsem0).wait()
        pltpu.make_async_copy(dsts_hbm.at[wid], ds_v, gsem1).wait()
        plsc.subcore_barrier()

        def fire_gathers(j0, bank, sem):
            for b in range(GRP):
                pltpu.async_copy(w_hbm.at[sg_v.at[j0 + b]],
                                 rows.at[bank * GRP + b], sem)

        def wait_gathers(j0, bank, sem):
            for b in range(GRP):
                pltpu.make_async_copy(w_hbm.at[sg_v.at[j0 + b]],
                                      rows.at[bank * GRP + b], sem).wait()

        def fire_scatters(j0, bank, sem):
            for b in range(GRP):
                pltpu.async_copy(rows.at[bank * GRP + b],
                                 acc_sh.at[ds_v.at[j0 + b]], sem, add=True)

        def wait_scatters(j0, bank, sem):
            for b in range(GRP):
                pltpu.make_async_copy(rows.at[bank * GRP + b],
                                      acc_sh.at[ds_v.at[j0 + b]], sem).wait()

        ngrp = NCH // GRP
        fire_gathers(0, 0, gsem0)

        def body(g, carry):
            t0 = 2 * g
            wait_gathers(t0 * GRP, 0, gsem0)
            fire_scatters(t0 * GRP, 0, ssem0)

            @pl.when(g > 0)
            def _():
                wait_scatters((t0 - 1) * GRP, 1, ssem1)

            fire_gathers((t0 + 1) * GRP, 1, gsem1)
            wait_gathers((t0 + 1) * GRP, 1, gsem1)
            fire_scatters((t0 + 1) * GRP, 1, ssem1)
            wait_scatters(t0 * GRP, 0, ssem0)

            @pl.when(t0 + 2 < ngrp)
            def _():
                fire_gathers((t0 + 2) * GRP, 0, gsem0)

            return carry

        lax.fori_loop(0, ngrp // 2, body, 0)
        wait_scatters((ngrp - 1) * GRP, 1, ssem1)
        plsc.subcore_barrier()
        pltpu.sync_copy(acc_sh.at[pl.ds(sid * RPT, RPT)],
                        out_hbm.at[cid, pl.ds(sid * RPT, RPT)])

    return k(w, srcg, dsts)


def _tc_pre(x, wcat, degpt):
    def body(x_ref, w_ref, dg_ref, xw_ref, wa_ref):
        xw = jnp.dot(x_ref[...], w_ref[...], preferred_element_type=_f32)
        deg = dg_ref[:, 0:1] + dg_ref[:, 1:2]
        dis = jnp.where(deg > 0, lax.rsqrt(jnp.maximum(deg, 1e-12)), 0.0)
        xw_ref[...] = xw
        wa_ref[...] = dis * xw[:, 2 * H:3 * H]

    return pl.pallas_call(
        body,
        grid=(NB,),
        in_specs=[
            pl.BlockSpec((RB, D), lambda i: (i, 0)),
            pl.BlockSpec((D, 3 * H), lambda i: (0, 0)),
            pl.BlockSpec((RB, 2), lambda i: (i, 0)),
        ],
        out_specs=[
            pl.BlockSpec((RB, 3 * H), lambda i: (i, 0)),
            pl.BlockSpec((RB, H), lambda i: (i, 0)),
        ],
        out_shape=[
            jax.ShapeDtypeStruct((N, 3 * H), _f32),
            jax.ShapeDtypeStruct((N, H), _f32),
        ],
    )(x, wcat, degpt)


def _tc_mid(xw, degpt, p):
    def body(xw_ref, dg_ref, p_ref, wc_ref):
        ps = p_ref[0] + p_ref[1]
        deg = dg_ref[:, 0:1] + dg_ref[:, 1:2]
        dis = jnp.where(deg > 0, lax.rsqrt(jnp.maximum(deg, 1e-12)), 0.0)
        wc_ref[...] = dis * (xw_ref[:, H:2 * H] - 2.0 * dis * ps)

    return pl.pallas_call(
        body,
        grid=(NB,),
        in_specs=[
            pl.BlockSpec((RB, 3 * H), lambda i: (i, 0)),
            pl.BlockSpec((RB, 2), lambda i: (i, 0)),
            pl.BlockSpec((2, RB, H), lambda i: (0, i, 0)),
        ],
        out_specs=pl.BlockSpec((RB, H), lambda i: (i, 0)),
        out_shape=jax.ShapeDtypeStruct((N, H), _f32),
    )(xw, degpt, p)


def _tc_layer(xw, degpt, p, b, w2cat):
    def body(xw_ref, dg_ref, p_ref, b_ref, w2_ref, xw2_ref, wa2_ref):
        ps = p_ref[0] + p_ref[1]
        deg = dg_ref[:, 0:1] + dg_ref[:, 1:2]
        dis = jnp.where(deg > 0, lax.rsqrt(jnp.maximum(deg, 1e-12)), 0.0)
        h = jnp.maximum(
            xw_ref[:, 0:H] - xw_ref[:, 2 * H:3 * H] - dis * ps + b_ref[...], 0.0)
        xw2 = jnp.dot(h, w2_ref[...], preferred_element_type=_f32)
        xw2_ref[...] = xw2
        wa2_ref[...] = dis * xw2[:, 2 * H:3 * H]

    return pl.pallas_call(
        body,
        grid=(NB,),
        in_specs=[
            pl.BlockSpec((RB, 3 * H), lambda i: (i, 0)),
            pl.BlockSpec((RB, 2), lambda i: (i, 0)),
            pl.BlockSpec((2, RB, H), lambda i: (0, i, 0)),
            pl.BlockSpec((1, H), lambda i: (0, 0)),
            pl.BlockSpec((H, 3 * H), lambda i: (0, 0)),
        ],
        out_specs=[
            pl.BlockSpec((RB, 3 * H), lambda i: (i, 0)),
            pl.BlockSpec((RB, H), lambda i: (i, 0)),
        ],
        out_shape=[
            jax.ShapeDtypeStruct((N, 3 * H), _f32),
            jax.ShapeDtypeStruct((N, H), _f32),
        ],
    )(xw, degpt, p, b, w2cat)


def _tc_final(xw2, aux, p, b, wfc, bfc):
    def body(xw_ref, aux_ref, p_ref, b_ref, wfc_ref, bfc_ref, out_ref,
             s_sum, s_cnt):
        i = pl.program_id(0)

        @pl.when(i == 0)
        def _():
            s_sum[...] = jnp.zeros((G, H), _f32)
            s_cnt[...] = jnp.zeros((G, 128), _f32)

        ps = p_ref[0] + p_ref[1]
        deg = aux_ref[:, 0:1] + aux_ref[:, 1:2]
        dis = jnp.where(deg > 0, lax.rsqrt(jnp.maximum(deg, 1e-12)), 0.0)
        h2 = jnp.maximum(
            xw_ref[:, 0:H] - xw_ref[:, 2 * H:3 * H] - dis * ps + b_ref[...], 0.0)
        bat = aux_ref[:, 2].astype(jnp.int32)
        onehot = (lax.broadcasted_iota(jnp.int32, (G, RB), 0)
                  == bat[None, :]).astype(_f32)
        s_sum[...] += jnp.dot(onehot, h2, preferred_element_type=_f32)
        s_cnt[...] += jnp.broadcast_to(
            jnp.sum(onehot, axis=1)[:, None], (G, 128))

        @pl.when(i == NB - 1)
        def _():
            pooled = s_sum[...] / jnp.maximum(s_cnt[:, 0:1], 1.0)
            out_ref[...] = (jnp.dot(pooled, wfc_ref[...],
                                    preferred_element_type=_f32) + bfc_ref[...])

    return pl.pallas_call(
        body,
        grid=(NB,),
        in_specs=[
            pl.BlockSpec((RB, 3 * H), lambda i: (i, 0)),
            pl.BlockSpec((RB, 3), lambda i: (i, 0)),
            pl.BlockSpec((2, RB, H), lambda i: (0, i, 0)),
            pl.BlockSpec((1, H), lambda i: (0, 0)),
            pl.BlockSpec((H, 1), lambda i: (0, 0)),
            pl.BlockSpec((1, 1), lambda i: (0, 0)),
        ],
        out_specs=pl.BlockSpec((G, 1), lambda i: (0, 0)),
        out_shape=jax.ShapeDtypeStruct((G, 1), _f32),
        scratch_shapes=[
            pltpu.VMEM((G, H), _f32),
            pltpu.VMEM((G, 128), _f32),
        ],
    )(xw2, aux, p, b, wfc, bfc)


def kernel(x, edge_index, batch, W1, b1, W2, b2, Wfc, bfc):
    src = edge_index[0].astype(jnp.int32)
    dst = edge_index[1].astype(jnp.int32)

    npad = EPAD - E
    padi = jnp.arange(npad, dtype=jnp.int32)
    srcg = jnp.concatenate([src, padi % N]).reshape(NW, NCH, CHUNK)
    srcs = jnp.concatenate([src, N + (padi % 16)]).reshape(NW, NCH, CHUNK)
    dsts = jnp.concatenate([dst, N + (padi % 16)]).reshape(NW, NCH, CHUNK)

    w1cat = jnp.concatenate([W1[0], W1[1], W1[2]], axis=1)
    w2cat = jnp.concatenate([W2[0], W2[1], W2[2]], axis=1)
    b1r = b1.reshape(1, H)
    b2r = b2.reshape(1, H)
    bfcr = bfc.reshape(1, 1)

    degp = _sc_deg(srcs)
    degpt = degp.T[:N]
    aux = jnp.concatenate(
        [degpt, batch.astype(_f32).reshape(N, 1)], axis=1)

    xw1, wa1 = _tc_pre(x, w1cat, degpt)
    p1 = _sc_matvec(wa1, srcg, dsts)
    wc1 = _tc_mid(xw1, degpt, p1)
    p2 = _sc_matvec(wc1, srcg, dsts)
    xw2, wa2 = _tc_layer(xw1, degpt, p2, b1r, w2cat)
    p3 = _sc_matvec(wa2, srcg, dsts)
    wc2 = _tc_mid(xw2, degpt, p3)
    p4 = _sc_matvec(wc2, srcg, dsts)
    out = _tc_final(xw2, aux, p4, b2r, Wfc, bfcr)
    return out[:, 0]


# speedup vs baseline: 1.0301x; 1.0093x over previous
import functools

import jax
import jax.numpy as jnp
from jax import lax
from jax.experimental import pallas as pl
from jax.experimental.pallas import tpu as pltpu
from jax.experimental.pallas import tpu_sc as plsc

N = 10000
E = 320000
D = 128
G = 8
H = 64

NW = 32
CHUNK = 128
GRP = 4
NCH = 80
EPAD = NW * NCH * CHUNK
NPAD = 10240
RPT = NPAD // 16
RB = 5000
NB = N // RB

_f32 = jnp.float32


def _sc_mesh():
    return plsc.VectorSubcoreMesh(core_axis_name="c", subcore_axis_name="s")


_SC_PARAMS = pltpu.CompilerParams(use_tc_tiling_on_sc=False)


def _sc_deg(srcs):
    @functools.partial(
        pl.kernel,
        out_type=jax.ShapeDtypeStruct((2, NPAD), _f32),
        mesh=_sc_mesh(),
        compiler_params=_SC_PARAMS,
        scratch_types=[
            pltpu.VMEM((NCH, CHUNK), jnp.int32),
            pltpu.VMEM((CHUNK,), _f32),
            pltpu.VMEM((RPT,), _f32),
            pltpu.VMEM_SHARED((NPAD,), _f32),
        ],
    )
    def k(srcs_hbm, out_hbm, si_v, ones_v, zb_v, dacc_sh):
        cid = lax.axis_index("c")
        sid = lax.axis_index("s")
        wid = sid * 2 + cid
        pltpu.sync_copy(srcs_hbm.at[wid], si_v)
        for t in range(CHUNK // 16):
            ones_v[pl.ds(t * 16, 16)] = jnp.ones((16,), _f32)

        def zinit(i, carry):
            zb_v[pl.ds(i * 16, 16)] = jnp.zeros((16,), _f32)
            return carry

        lax.fori_loop(0, RPT // 16, zinit, 0)
        pltpu.sync_copy(zb_v, dacc_sh.at[pl.ds(sid * RPT, RPT)])
        plsc.subcore_barrier()

        def body(j, carry):
            pltpu.sync_copy(ones_v, dacc_sh.at[si_v.at[j]], add=True)
            return carry

        lax.fori_loop(0, NCH, body, 0)
        plsc.subcore_barrier()
        pltpu.sync_copy(dacc_sh.at[pl.ds(sid * RPT, RPT)],
                        out_hbm.at[cid, pl.ds(sid * RPT, RPT)])

    return k(srcs)


def _sc_matvec(w, srcg, dsts):
    @functools.partial(
        pl.kernel,
        out_type=jax.ShapeDtypeStruct((2, NPAD, H), _f32),
        mesh=_sc_mesh(),
        compiler_params=_SC_PARAMS,
        scratch_types=[
            pltpu.VMEM((NCH, CHUNK), jnp.int32),
            pltpu.VMEM((NCH, CHUNK), jnp.int32),
            pltpu.VMEM((2 * GRP, CHUNK, H), _f32),
            pltpu.VMEM_SHARED((NPAD, H), _f32),
            pltpu.SemaphoreType.DMA,
            pltpu.SemaphoreType.DMA,
            pltpu.SemaphoreType.DMA,
            pltpu.SemaphoreType.DMA,
        ],
    )
    def k(w_hbm, srcg_hbm, dsts_hbm, out_hbm, sg_v, ds_v, rows, acc_sh,
          gsem0, gsem1, ssem0, ssem1):
        cid = lax.axis_index("c")
        sid = lax.axis_index("s")
        wid = sid * 2 + cid
        gsems = (gsem0, gsem1)
        ssems = (ssem0, ssem1)
        pltpu.async_copy(srcg_hbm.at[wid], sg_v, gsem0)
        pltpu.async_copy(dsts_hbm.at[wid], ds_v, gsem1)

        def zinit(i, carry):
            r = i // (H // 16)
            c = i % (H // 16)
            rows[0, r, pl.ds(c * 16, 16)] = jnp.zeros((16,), _f32)
            return carry

        lax.fori_loop(0, CHUNK * (H // 16), zinit, 0)
        for t in range(RPT // CHUNK):
            pltpu.sync_copy(rows.at[0],
                            acc_sh.at[pl.ds(sid * RPT + t * CHUNK, CHUNK)])
        pltpu.make_async_copy(srcg_hbm.at[wid], sg_v, gsem0).wait()
        pltpu.make_async_copy(dsts_hbm.at[wid], ds_v, gsem1).wait()
        plsc.subcore_barrier()

        def fire_gathers(j0, bank, sem):
            for b in range(GRP):
                pltpu.async_copy(w_hbm.at[sg_v.at[j0 + b]],
                                 rows.at[bank * GRP + b], sem)

        def wait_gathers(j0, bank, sem):
            for b in range(GRP):
                pltpu.make_async_copy(w_hbm.at[sg_v.at[j0 + b]],
                                      rows.at[bank * GRP + b], sem).wait()

        def fire_scatters(j0, bank, sem):
            for b in range(GRP):
                pltpu.async_copy(rows.at[bank * GRP + b],
                                 acc_sh.at[ds_v.at[j0 + b]], sem, add=True)

        def wait_scatters(j0, bank, sem):
            for b in range(GRP):
                pltpu.make_async_copy(rows.at[bank * GRP + b],
                                      acc_sh.at[ds_v.at[j0 + b]], sem).wait()

        ngrp = NCH // GRP
        fire_gathers(0, 0, gsem0)

        def body(g, carry):
            t0 = 2 * g
            wait_gathers(t0 * GRP, 0, gsem0)
            fire_scatters(t0 * GRP, 0, ssem0)

            @pl.when(g > 0)
            def _():
                wait_scatters((t0 - 1) * GRP, 1, ssem1)

            fire_gathers((t0 + 1) * GRP, 1, gsem1)
            wait_gathers((t0 + 1) * GRP, 1, gsem1)
            fire_scatters((t0 + 1) * GRP, 1, ssem1)
            wait_scatters(t0 * GRP, 0, ssem0)

            @pl.when(t0 + 2 < ngrp)
            def _():
                fire_gathers((t0 + 2) * GRP, 0, gsem0)

            return carry

        lax.fori_loop(0, ngrp // 2, body, 0)
        wait_scatters((ngrp - 1) * GRP, 1, ssem1)
        plsc.subcore_barrier()
        pltpu.sync_copy(acc_sh.at[pl.ds(sid * RPT, RPT)],
                        out_hbm.at[cid, pl.ds(sid * RPT, RPT)])

    return k(w, srcg, dsts)


def _tc_pre(x, wcat, degpt):
    def body(x_ref, w_ref, dg_ref, xw_ref, wa_ref):
        xw = jnp.dot(x_ref[...], w_ref[...], preferred_element_type=_f32)
        deg = dg_ref[:, 0:1] + dg_ref[:, 1:2]
        dis = jnp.where(deg > 0, lax.rsqrt(jnp.maximum(deg, 1e-12)), 0.0)
        xw_ref[...] = xw
        wa_ref[...] = dis * xw[:, 2 * H:3 * H]

    return pl.pallas_call(
        body,
        grid=(NB,),
        in_specs=[
            pl.BlockSpec((RB, D), lambda i: (i, 0)),
            pl.BlockSpec((D, 3 * H), lambda i: (0, 0)),
            pl.BlockSpec((RB, 2), lambda i: (i, 0)),
        ],
        out_specs=[
            pl.BlockSpec((RB, 3 * H), lambda i: (i, 0)),
            pl.BlockSpec((RB, H), lambda i: (i, 0)),
        ],
        out_shape=[
            jax.ShapeDtypeStruct((N, 3 * H), _f32),
            jax.ShapeDtypeStruct((N, H), _f32),
        ],
    )(x, wcat, degpt)


def _tc_mid(xw, degpt, p):
    def body(xw_ref, dg_ref, p_ref, wc_ref):
        ps = p_ref[0] + p_ref[1]
        deg = dg_ref[:, 0:1] + dg_ref[:, 1:2]
        dis = jnp.where(deg > 0, lax.rsqrt(jnp.maximum(deg, 1e-12)), 0.0)
        wc_ref[...] = dis * (xw_ref[:, H:2 * H] - 2.0 * dis * ps)

    return pl.pallas_call(
        body,
        grid=(NB,),
        in_specs=[
            pl.BlockSpec((RB, 3 * H), lambda i: (i, 0)),
            pl.BlockSpec((RB, 2), lambda i: (i, 0)),
            pl.BlockSpec((2, RB, H), lambda i: (0, i, 0)),
        ],
        out_specs=pl.BlockSpec((RB, H), lambda i: (i, 0)),
        out_shape=jax.ShapeDtypeStruct((N, H), _f32),
    )(xw, degpt, p)


def _tc_layer(xw, degpt, p, b, w2cat):
    def body(xw_ref, dg_ref, p_ref, b_ref, w2_ref, xw2_ref, wa2_ref):
        ps = p_ref[0] + p_ref[1]
        deg = dg_ref[:, 0:1] + dg_ref[:, 1:2]
        dis = jnp.where(deg > 0, lax.rsqrt(jnp.maximum(deg, 1e-12)), 0.0)
        h = jnp.maximum(
            xw_ref[:, 0:H] - xw_ref[:, 2 * H:3 * H] - dis * ps + b_ref[...], 0.0)
        xw2 = jnp.dot(h, w2_ref[...], preferred_element_type=_f32)
        xw2_ref[...] = xw2
        wa2_ref[...] = dis * xw2[:, 2 * H:3 * H]

    return pl.pallas_call(
        body,
        grid=(NB,),
        in_specs=[
            pl.BlockSpec((RB, 3 * H), lambda i: (i, 0)),
            pl.BlockSpec((RB, 2), lambda i: (i, 0)),
            pl.BlockSpec((2, RB, H), lambda i: (0, i, 0)),
            pl.BlockSpec((1, H), lambda i: (0, 0)),
            pl.BlockSpec((H, 3 * H), lambda i: (0, 0)),
        ],
        out_specs=[
            pl.BlockSpec((RB, 3 * H), lambda i: (i, 0)),
            pl.BlockSpec((RB, H), lambda i: (i, 0)),
        ],
        out_shape=[
            jax.ShapeDtypeStruct((N, 3 * H), _f32),
            jax.ShapeDtypeStruct((N, H), _f32),
        ],
    )(xw, degpt, p, b, w2cat)


def _tc_final(xw2, aux, p, b, wfc, bfc):
    def body(xw_ref, aux_ref, p_ref, b_ref, wfc_ref, bfc_ref, out_ref,
             s_sum, s_cnt):
        i = pl.program_id(0)

        @pl.when(i == 0)
        def _():
            s_sum[...] = jnp.zeros((G, H), _f32)
            s_cnt[...] = jnp.zeros((G, 128), _f32)

        ps = p_ref[0] + p_ref[1]
        deg = aux_ref[:, 0:1] + aux_ref[:, 1:2]
        dis = jnp.where(deg > 0, lax.rsqrt(jnp.maximum(deg, 1e-12)), 0.0)
        h2 = jnp.maximum(
            xw_ref[:, 0:H] - xw_ref[:, 2 * H:3 * H] - dis * ps + b_ref[...], 0.0)
        bat = aux_ref[:, 2].astype(jnp.int32)
        onehot = (lax.broadcasted_iota(jnp.int32, (G, RB), 0)
                  == bat[None, :]).astype(_f32)
        s_sum[...] += jnp.dot(onehot, h2, preferred_element_type=_f32)
        s_cnt[...] += jnp.broadcast_to(
            jnp.sum(onehot, axis=1)[:, None], (G, 128))

        @pl.when(i == NB - 1)
        def _():
            pooled = s_sum[...] / jnp.maximum(s_cnt[:, 0:1], 1.0)
            out_ref[...] = (jnp.dot(pooled, wfc_ref[...],
                                    preferred_element_type=_f32) + bfc_ref[...])

    return pl.pallas_call(
        body,
        grid=(NB,),
        in_specs=[
            pl.BlockSpec((RB, 3 * H), lambda i: (i, 0)),
            pl.BlockSpec((RB, 3), lambda i: (i, 0)),
            pl.BlockSpec((2, RB, H), lambda i: (0, i, 0)),
            pl.BlockSpec((1, H), lambda i: (0, 0)),
            pl.BlockSpec((H, 1), lambda i: (0, 0)),
            pl.BlockSpec((1, 1), lambda i: (0, 0)),
        ],
        out_specs=pl.BlockSpec((G, 1), lambda i: (0, 0)),
        out_shape=jax.ShapeDtypeStruct((G, 1), _f32),
        scratch_shapes=[
            pltpu.VMEM((G, H), _f32),
            pltpu.VMEM((G, 128), _f32),
        ],
    )(xw2, aux, p, b, wfc, bfc)


def kernel(x, edge_index, batch, W1, b1, W2, b2, Wfc, bfc):
    src = edge_index[0].astype(jnp.int32)
    dst = edge_index[1].astype(jnp.int32)

    npad = EPAD - E
    padi = jnp.arange(npad, dtype=jnp.int32)
    srcg = jnp.concatenate([src, padi % N]).reshape(NW, NCH, CHUNK)
    srcs = jnp.concatenate([src, N + (padi % 16)]).reshape(NW, NCH, CHUNK)
    dsts = jnp.concatenate([dst, N + (padi % 16)]).reshape(NW, NCH, CHUNK)

    w1cat = jnp.concatenate([W1[0], W1[1], W1[2]], axis=1)
    w2cat = jnp.concatenate([W2[0], W2[1], W2[2]], axis=1)
    b1r = b1.reshape(1, H)
    b2r = b2.reshape(1, H)
    bfcr = bfc.reshape(1, 1)

    degp = _sc_deg(srcs)
    degpt = degp.T[:N]
    aux = jnp.concatenate(
        [degpt, batch.astype(_f32).reshape(N, 1)], axis=1)

    xw1, wa1 = _tc_pre(x, w1cat, degpt)
    p1 = _sc_matvec(wa1, srcg, dsts)
    wc1 = _tc_mid(xw1, degpt, p1)
    p2 = _sc_matvec(wc1, srcg, dsts)
    xw2, wa2 = _tc_layer(xw1, degpt, p2, b1r, w2cat)
    p3 = _sc_matvec(wa2, srcg, dsts)
    wc2 = _tc_mid(xw2, degpt, p3)
    p4 = _sc_matvec(wc2, srcg, dsts)
    out = _tc_final(xw2, aux, p4, b2r, Wfc, bfcr)
    return out[:, 0]


# speedup vs baseline: 1.0557x; 1.0249x over previous
import functools

import jax
import jax.numpy as jnp
from jax import lax
from jax.experimental import pallas as pl
from jax.experimental.pallas import tpu as pltpu
from jax.experimental.pallas import tpu_sc as plsc

N = 10000
E = 320000
D = 128
G = 8
H = 64

NW = 32
CHUNK = 128
GRP = 4
NCH = 80
EPAD = NW * NCH * CHUNK
NPAD = 10240
RPT = NPAD // 16
RB = 5000
NB = N // RB

_f32 = jnp.float32


def _sc_mesh():
    return plsc.VectorSubcoreMesh(core_axis_name="c", subcore_axis_name="s")


_SC_PARAMS = pltpu.CompilerParams(use_tc_tiling_on_sc=False)


def _sc_deg(srcs):
    @functools.partial(
        pl.kernel,
        out_type=jax.ShapeDtypeStruct((2, NPAD), _f32),
        mesh=_sc_mesh(),
        compiler_params=_SC_PARAMS,
        scratch_types=[
            pltpu.VMEM((NCH, CHUNK), jnp.int32),
            pltpu.VMEM((CHUNK,), _f32),
            pltpu.VMEM((RPT,), _f32),
            pltpu.VMEM_SHARED((NPAD,), _f32),
        ],
    )
    def k(srcs_hbm, out_hbm, si_v, ones_v, zb_v, dacc_sh):
        cid = lax.axis_index("c")
        sid = lax.axis_index("s")
        wid = sid * 2 + cid
        pltpu.sync_copy(srcs_hbm.at[wid], si_v)
        for t in range(CHUNK // 16):
            ones_v[pl.ds(t * 16, 16)] = jnp.ones((16,), _f32)

        def zinit(i, carry):
            zb_v[pl.ds(i * 16, 16)] = jnp.zeros((16,), _f32)
            return carry

        lax.fori_loop(0, RPT // 16, zinit, 0)
        pltpu.sync_copy(zb_v, dacc_sh.at[pl.ds(sid * RPT, RPT)])
        plsc.subcore_barrier()

        def body(j, carry):
            pltpu.sync_copy(ones_v, dacc_sh.at[si_v.at[j]], add=True)
            return carry

        lax.fori_loop(0, NCH, body, 0)
        plsc.subcore_barrier()
        pltpu.sync_copy(dacc_sh.at[pl.ds(sid * RPT, RPT)],
                        out_hbm.at[cid, pl.ds(sid * RPT, RPT)])

    return k(srcs)


def _sc_matvec(w, srcg, dsts):
    @functools.partial(
        pl.kernel,
        out_type=jax.ShapeDtypeStruct((2, NPAD, H), _f32),
        mesh=_sc_mesh(),
        compiler_params=_SC_PARAMS,
        scratch_types=[
            pltpu.VMEM((NCH, CHUNK), jnp.int32),
            pltpu.VMEM((NCH, CHUNK), jnp.int32),
            pltpu.VMEM((2 * GRP, CHUNK, H), _f32),
            pltpu.VMEM_SHARED((NPAD, H), _f32),
            pltpu.SemaphoreType.DMA,
            pltpu.SemaphoreType.DMA,
            pltpu.SemaphoreType.DMA,
            pltpu.SemaphoreType.DMA,
        ],
    )
    def k(w_hbm, srcg_hbm, dsts_hbm, out_hbm, sg_v, ds_v, rows, acc_sh,
          gsem0, gsem1, ssem0, ssem1):
        cid = lax.axis_index("c")
        sid = lax.axis_index("s")
        wid = sid * 2 + cid
        gsems = (gsem0, gsem1)
        ssems = (ssem0, ssem1)
        pltpu.async_copy(srcg_hbm.at[wid], sg_v, gsem0)
        pltpu.async_copy(dsts_hbm.at[wid], ds_v, gsem1)

        def zinit(i, carry):
            r = i // (H // 16)
            c = i % (H // 16)
            rows[0, r, pl.ds(c * 16, 16)] = jnp.zeros((16,), _f32)
            return carry

        lax.fori_loop(0, CHUNK * (H // 16), zinit, 0)
        for t in range(RPT // CHUNK):
            pltpu.sync_copy(rows.at[0],
                            acc_sh.at[pl.ds(sid * RPT + t * CHUNK, CHUNK)])
        pltpu.make_async_copy(srcg_hbm.at[wid], sg_v, gsem0).wait()
        pltpu.make_async_copy(dsts_hbm.at[wid], ds_v, gsem1).wait()
        plsc.subcore_barrier()

        def fire_gathers(j0, bank, sem):
            for b in range(GRP):
                pltpu.async_copy(w_hbm.at[sg_v.at[j0 + b]],
                                 rows.at[bank * GRP + b], sem)

        def wait_gathers(j0, bank, sem):
            for b in range(GRP):
                pltpu.make_async_copy(w_hbm.at[sg_v.at[j0 + b]],
                                      rows.at[bank * GRP + b], sem).wait()

        def fire_scatters(j0, bank, sem):
            for b in range(GRP):
                pltpu.async_copy(rows.at[bank * GRP + b],
                                 acc_sh.at[ds_v.at[j0 + b]], sem, add=True)

        def wait_scatters(j0, bank, sem):
            for b in range(GRP):
                pltpu.make_async_copy(rows.at[bank * GRP + b],
                                      acc_sh.at[ds_v.at[j0 + b]], sem).wait()

        ngrp = NCH // GRP
        fire_gathers(0, 0, gsem0)

        def body(g, carry):
            t0 = 2 * g
            wait_gathers(t0 * GRP, 0, gsem0)
            fire_scatters(t0 * GRP, 0, ssem0)

            @pl.when(g > 0)
            def _():
                wait_scatters((t0 - 1) * GRP, 1, ssem1)

            fire_gathers((t0 + 1) * GRP, 1, gsem1)
            wait_gathers((t0 + 1) * GRP, 1, gsem1)
            fire_scatters((t0 + 1) * GRP, 1, ssem1)
            wait_scatters(t0 * GRP, 0, ssem0)

            @pl.when(t0 + 2 < ngrp)
            def _():
                fire_gathers((t0 + 2) * GRP, 0, gsem0)

            return carry

        lax.fori_loop(0, ngrp // 2, body, 0)
        wait_scatters((ngrp - 1) * GRP, 1, ssem1)
        plsc.subcore_barrier()
        pltpu.sync_copy(acc_sh.at[pl.ds(sid * RPT, RPT)],
                        out_hbm.at[cid, pl.ds(sid * RPT, RPT)])

    return k(w, srcg, dsts)


def _tc_pre(x, wcat, degpt):
    def body(x_ref, w_ref, dg_ref, xwd_ref, xwm_ref, wa_ref):
        xw = jnp.dot(x_ref[...], w_ref[...], preferred_element_type=_f32)
        deg = dg_ref[:, 0:1] + dg_ref[:, 1:2]
        dis = jnp.where(deg > 0, lax.rsqrt(jnp.maximum(deg, 1e-12)), 0.0)
        xwd_ref[...] = xw[:, 0:H] - xw[:, 2 * H:3 * H]
        xwm_ref[...] = xw[:, H:2 * H]
        wa_ref[...] = dis * xw[:, 2 * H:3 * H]

    return pl.pallas_call(
        body,
        grid=(NB,),
        in_specs=[
            pl.BlockSpec((RB, D), lambda i: (i, 0)),
            pl.BlockSpec((D, 3 * H), lambda i: (0, 0)),
            pl.BlockSpec((RB, 2), lambda i: (i, 0)),
        ],
        out_specs=[
            pl.BlockSpec((RB, H), lambda i: (i, 0)),
            pl.BlockSpec((RB, H), lambda i: (i, 0)),
            pl.BlockSpec((RB, H), lambda i: (i, 0)),
        ],
        out_shape=[
            jax.ShapeDtypeStruct((N, H), _f32),
            jax.ShapeDtypeStruct((N, H), _f32),
            jax.ShapeDtypeStruct((N, H), _f32),
        ],
    )(x, wcat, degpt)


def _tc_mid(xwm, degpt, p):
    def body(xw_ref, dg_ref, p_ref, wc_ref):
        ps = p_ref[0] + p_ref[1]
        deg = dg_ref[:, 0:1] + dg_ref[:, 1:2]
        dis = jnp.where(deg > 0, lax.rsqrt(jnp.maximum(deg, 1e-12)), 0.0)
        wc_ref[...] = dis * (xw_ref[...] - 2.0 * dis * ps)

    return pl.pallas_call(
        body,
        grid=(NB,),
        in_specs=[
            pl.BlockSpec((RB, H), lambda i: (i, 0)),
            pl.BlockSpec((RB, 2), lambda i: (i, 0)),
            pl.BlockSpec((2, RB, H), lambda i: (0, i, 0)),
        ],
        out_specs=pl.BlockSpec((RB, H), lambda i: (i, 0)),
        out_shape=jax.ShapeDtypeStruct((N, H), _f32),
    )(xwm, degpt, p)


def _tc_layer(xwd, degpt, p, b, w2cat):
    def body(xw_ref, dg_ref, p_ref, b_ref, w2_ref, xwd2_ref, xwm2_ref,
             wa2_ref):
        ps = p_ref[0] + p_ref[1]
        deg = dg_ref[:, 0:1] + dg_ref[:, 1:2]
        dis = jnp.where(deg > 0, lax.rsqrt(jnp.maximum(deg, 1e-12)), 0.0)
        h = jnp.maximum(xw_ref[...] - dis * ps + b_ref[...], 0.0)
        xw2 = jnp.dot(h, w2_ref[...], preferred_element_type=_f32)
        xwd2_ref[...] = xw2[:, 0:H] - xw2[:, 2 * H:3 * H]
        xwm2_ref[...] = xw2[:, H:2 * H]
        wa2_ref[...] = dis * xw2[:, 2 * H:3 * H]

    return pl.pallas_call(
        body,
        grid=(NB,),
        in_specs=[
            pl.BlockSpec((RB, H), lambda i: (i, 0)),
            pl.BlockSpec((RB, 2), lambda i: (i, 0)),
            pl.BlockSpec((2, RB, H), lambda i: (0, i, 0)),
            pl.BlockSpec((1, H), lambda i: (0, 0)),
            pl.BlockSpec((H, 3 * H), lambda i: (0, 0)),
        ],
        out_specs=[
            pl.BlockSpec((RB, H), lambda i: (i, 0)),
            pl.BlockSpec((RB, H), lambda i: (i, 0)),
            pl.BlockSpec((RB, H), lambda i: (i, 0)),
        ],
        out_shape=[
            jax.ShapeDtypeStruct((N, H), _f32),
            jax.ShapeDtypeStruct((N, H), _f32),
            jax.ShapeDtypeStruct((N, H), _f32),
        ],
    )(xwd, degpt, p, b, w2cat)


def _tc_final(xwd2, aux, p, b, wfc, bfc):
    def body(xw_ref, aux_ref, p_ref, b_ref, wfc_ref, bfc_ref, out_ref,
             s_sum, s_cnt):
        i = pl.program_id(0)

        @pl.when(i == 0)
        def _():
            s_sum[...] = jnp.zeros((G, H), _f32)
            s_cnt[...] = jnp.zeros((G, 128), _f32)

        ps = p_ref[0] + p_ref[1]
        deg = aux_ref[:, 0:1] + aux_ref[:, 1:2]
        dis = jnp.where(deg > 0, lax.rsqrt(jnp.maximum(deg, 1e-12)), 0.0)
        h2 = jnp.maximum(xw_ref[...] - dis * ps + b_ref[...], 0.0)
        bat = aux_ref[:, 2].astype(jnp.int32)
        onehot = (lax.broadcasted_iota(jnp.int32, (G, RB), 0)
                  == bat[None, :]).astype(_f32)
        s_sum[...] += jnp.dot(onehot, h2, preferred_element_type=_f32)
        s_cnt[...] += jnp.broadcast_to(
            jnp.sum(onehot, axis=1)[:, None], (G, 128))

        @pl.when(i == NB - 1)
        def _():
            pooled = s_sum[...] / jnp.maximum(s_cnt[:, 0:1], 1.0)
            out_ref[...] = (jnp.dot(pooled, wfc_ref[...],
                                    preferred_element_type=_f32) + bfc_ref[...])

    return pl.pallas_call(
        body,
        grid=(NB,),
        in_specs=[
            pl.BlockSpec((RB, H), lambda i: (i, 0)),
            pl.BlockSpec((RB, 3), lambda i: (i, 0)),
            pl.BlockSpec((2, RB, H), lambda i: (0, i, 0)),
            pl.BlockSpec((1, H), lambda i: (0, 0)),
            pl.BlockSpec((H, 1), lambda i: (0, 0)),
            pl.BlockSpec((1, 1), lambda i: (0, 0)),
        ],
        out_specs=pl.BlockSpec((G, 1), lambda i: (0, 0)),
        out_shape=jax.ShapeDtypeStruct((G, 1), _f32),
        scratch_shapes=[
            pltpu.VMEM((G, H), _f32),
            pltpu.VMEM((G, 128), _f32),
        ],
    )(xwd2, aux, p, b, wfc, bfc)


def kernel(x, edge_index, batch, W1, b1, W2, b2, Wfc, bfc):
    src = edge_index[0].astype(jnp.int32)
    dst = edge_index[1].astype(jnp.int32)

    npad = EPAD - E
    padi = jnp.arange(npad, dtype=jnp.int32)
    srcg = jnp.concatenate([src, padi % N]).reshape(NW, NCH, CHUNK)
    srcs = jnp.concatenate([src, N + (padi % 16)]).reshape(NW, NCH, CHUNK)
    dsts = jnp.concatenate([dst, N + (padi % 16)]).reshape(NW, NCH, CHUNK)

    w1cat = jnp.concatenate([W1[0], W1[1], W1[2]], axis=1)
    w2cat = jnp.concatenate([W2[0], W2[1], W2[2]], axis=1)
    b1r = b1.reshape(1, H)
    b2r = b2.reshape(1, H)
    bfcr = bfc.reshape(1, 1)

    degp = _sc_deg(srcs)
    degpt = degp.T[:N]
    aux = jnp.concatenate(
        [degpt, batch.astype(_f32).reshape(N, 1)], axis=1)

    xwd1, xwm1, wa1 = _tc_pre(x, w1cat, degpt)
    p1 = _sc_matvec(wa1, srcg, dsts)
    wc1 = _tc_mid(xwm1, degpt, p1)
    p2 = _sc_matvec(wc1, srcg, dsts)
    xwd2, xwm2, wa2 = _tc_layer(xwd1, degpt, p2, b1r, w2cat)
    p3 = _sc_matvec(wa2, srcg, dsts)
    wc2 = _tc_mid(xwm2, degpt, p3)
    p4 = _sc_matvec(wc2, srcg, dsts)
    out = _tc_final(xwd2, aux, p4, b2r, Wfc, bfcr)
    return out[:, 0]
